# Initial kernel scaffold; baseline (speedup 1.0000x reference)
import jax
import jax.numpy as jnp
from jax.experimental import pallas as pl


def kernel(x, pe):
    raise NotImplementedError("write your pallas kernel here")


# speedup vs baseline: 1.4637x; 1.4637x over previous
import jax
import jax.numpy as jnp
from jax.experimental import pallas as pl


def _add_block(x_ref, pe_ref, o_ref):
    o_ref[...] = x_ref[...] + pe_ref[...]


def kernel(x, pe):
    B, L, D = x.shape
    LB = 512
    grid = (B, L // LB)
    return pl.pallas_call(
        _add_block,
        grid=grid,
        in_specs=[
            pl.BlockSpec((1, LB, D), lambda b, j: (b, j, 0)),
            pl.BlockSpec((LB, D), lambda b, j: (j, 0)),
        ],
        out_specs=pl.BlockSpec((1, LB, D), lambda b, j: (b, j, 0)),
        out_shape=jax.ShapeDtypeStruct((B, L, D), x.dtype),
    )(x, pe[:L])


# speedup vs baseline: 1.6977x; 1.1599x over previous
import jax
import jax.numpy as jnp
from jax.experimental import pallas as pl


def _add_block(x_ref, pe_ref, o_ref):
    o_ref[...] = x_ref[...] + pe_ref[...]


def kernel(x, pe):
    B, L, D = x.shape
    LB = 512
    grid = (L // LB, B)
    return pl.pallas_call(
        _add_block,
        grid=grid,
        in_specs=[
            pl.BlockSpec((1, LB, D), lambda j, b: (b, j, 0)),
            pl.BlockSpec((LB, D), lambda j, b: (j, 0)),
        ],
        out_specs=pl.BlockSpec((1, LB, D), lambda j, b: (b, j, 0)),
        out_shape=jax.ShapeDtypeStruct((B, L, D), x.dtype),
    )(x, pe[:L])
